# initial kernel scaffold (unmeasured)
import jax
import jax.numpy as jnp
from jax import lax
from jax.experimental import pallas as pl
from jax.experimental.pallas import tpu as pltpu

B, H, D, BS = 8, 8, 128, 16
NB = 512


def kernel(Q, K, V, bt, lens):
    npages = K.shape[0]
    nkeys = npages * BS

    def body(q_ref, k_ref, v_ref, bt_ref, lens_ref, out_ref,
             comm_ref, send_sem, recv_sem):
        my_x = lax.axis_index("x")
        my_y = lax.axis_index("y")
        my_z = lax.axis_index("z")
        partner = (my_x, 1 - my_y, my_z)

        barrier = pltpu.get_barrier_semaphore()
        pl.semaphore_signal(barrier, inc=1, device_id=partner,
                            device_id_type=pl.DeviceIdType.MESH)
        pl.semaphore_wait(barrier, 1)

        page_off = my_y * npages
        bt_v = bt_ref[:, :]
        lens_v = lens_ref[:, :]
        slot = lax.broadcasted_iota(jnp.int32, (B, NB, 1), 1)
        valid = slot < lens_v[:, :, None]
        pids = lax.broadcasted_iota(jnp.int32, (B, NB, npages), 2) + page_off
        match = jnp.logical_and(bt_v[:, :, None] == pids, valid)
        counts = jnp.sum(match.astype(jnp.float32), axis=1)
        w = jnp.repeat(counts, BS, axis=1)

        scale = D ** -0.5
        accs = []
        ms = []
        ls = []
        for h in range(H):
            qh = q_ref[:, 0, h, :]
            kh = k_ref[:, :, h, :].reshape(nkeys, D)
            vh = v_ref[:, :, h, :].reshape(nkeys, D)
            s = lax.dot_general(
                qh, kh, (((1,), (1,)), ((), ())),
                preferred_element_type=jnp.float32) * scale
            m_h = jnp.max(s, axis=1, keepdims=True)
            p = jnp.exp(s - m_h) * w
            l_h = jnp.sum(p, axis=1, keepdims=True)
            acc_h = lax.dot_general(
                p, vh, (((1,), (0,)), ((), ())),
                preferred_element_type=jnp.float32)
            accs.append(acc_h)
            ms.append(m_h)
            ls.append(l_h)

        acc = jnp.stack(accs, axis=1)
        m = jnp.concatenate(ms, axis=1)
        l = jnp.concatenate(ls, axis=1)

        packed = jnp.concatenate(
            [acc,
             jnp.broadcast_to(m[:, :, None], (B, H, D)),
             jnp.broadcast_to(l[:, :, None], (B, H, D))], axis=2)
        comm_ref[0] = packed

        rdma = pltpu.make_async_remote_copy(
            src_ref=comm_ref.at[0],
            dst_ref=comm_ref.at[1],
            send_sem=send_sem,
            recv_sem=recv_sem,
            device_id=partner,
            device_id_type=pl.DeviceIdType.MESH,
        )
        rdma.start()
        rdma.wait()

        r = comm_ref[1]
        acc2 = r[:, :, :D]
        m2 = r[:, :, D]
        l2 = r[:, :, 2 * D]

        m12 = jnp.maximum(m, m2)
        a1 = jnp.exp(m - m12)
        a2 = jnp.exp(m2 - m12)
        lsum = a1 * l + a2 * l2
        o = (a1[:, :, None] * acc + a2[:, :, None] * acc2) / lsum[:, :, None]
        out_ref[:, :, :, :] = o[:, None, :, :]

    return pl.pallas_call(
        body,
        out_shape=jax.ShapeDtypeStruct((B, 1, H, D), jnp.float32),
        in_specs=[pl.BlockSpec(memory_space=pltpu.VMEM)] * 5,
        out_specs=pl.BlockSpec(memory_space=pltpu.VMEM),
        scratch_shapes=[
            pltpu.VMEM((2, B, H, 3 * D), jnp.float32),
            pltpu.SemaphoreType.DMA,
            pltpu.SemaphoreType.DMA,
        ],
        compiler_params=pltpu.CompilerParams(collective_id=0),
    )(Q, K, V, bt, lens.reshape(B, 1))


# baseline (device time: 88336 ns/iter reference)
import jax
import jax.numpy as jnp
from jax import lax
from jax.experimental import pallas as pl
from jax.experimental.pallas import tpu as pltpu

B, H, D, BS = 8, 8, 128, 16
NB = 512
CP = 64
CK = CP * BS

NEG_INF = -1e30


def kernel(Q, K, V, bt, lens):
    npages = K.shape[0]
    nsteps = npages // CP

    def body(q_ref, k_ref, v_ref, bt_ref, lens_ref, out_ref,
             comm_ref, acc_scr, m_scr, l_scr, send_sem, recv_sem):
        c = pl.program_id(0)
        my_x = lax.axis_index("x")
        my_y = lax.axis_index("y")
        my_z = lax.axis_index("z")
        partner = (my_x, 1 - my_y, my_z)

        @pl.when(c == 0)
        def _init():
            m_scr[...] = jnp.full(m_scr.shape, NEG_INF, jnp.float32)
            l_scr[...] = jnp.zeros_like(l_scr)
            acc_scr[...] = jnp.zeros_like(acc_scr)

        page_off = my_y * npages + c * CP
        bt_v = bt_ref[:, :]
        lens_v = lens_ref[:, :]
        slot = lax.broadcasted_iota(jnp.int32, (B, NB, 1), 1)
        valid = slot < lens_v[:, :, None]
        pids = lax.broadcasted_iota(jnp.int32, (B, NB, CP), 2) + page_off
        match = jnp.logical_and(bt_v[:, :, None] == pids, valid)
        counts = jnp.sum(match.astype(jnp.float32), axis=1)
        w = jnp.repeat(counts, BS, axis=1)

        scale = D ** -0.5
        for h in range(H):
            qh = q_ref[:, 0, h, :]
            kh = k_ref[:, :, h, :].reshape(CK, D)
            vh = v_ref[:, :, h, :].reshape(CK, D)
            s = lax.dot_general(
                qh, kh, (((1,), (1,)), ((), ())),
                preferred_element_type=jnp.float32) * scale
            m_old = m_scr[h]
            m_new = jnp.maximum(m_old, jnp.max(s, axis=1, keepdims=True))
            e = jnp.exp(s - m_new) * w
            alpha = jnp.exp(m_old - m_new)
            pv = lax.dot_general(
                e, vh, (((1,), (0,)), ((), ())),
                preferred_element_type=jnp.float32)
            acc_scr[h] = alpha * acc_scr[h] + pv
            l_scr[h] = alpha * l_scr[h] + jnp.sum(e, axis=1, keepdims=True)
            m_scr[h] = m_new

        @pl.when(c == nsteps - 1)
        def _exchange():
            acc = jnp.stack([acc_scr[h] for h in range(H)], axis=1)
            m = jnp.concatenate([m_scr[h] for h in range(H)], axis=1)
            l = jnp.concatenate([l_scr[h] for h in range(H)], axis=1)

            comm_ref[0] = jnp.concatenate(
                [acc,
                 jnp.broadcast_to(m[:, :, None], (B, H, D)),
                 jnp.broadcast_to(l[:, :, None], (B, H, D))], axis=2)

            barrier = pltpu.get_barrier_semaphore()
            pl.semaphore_signal(barrier, inc=1, device_id=partner,
                                device_id_type=pl.DeviceIdType.MESH)
            pl.semaphore_wait(barrier, 1)

            rdma = pltpu.make_async_remote_copy(
                src_ref=comm_ref.at[0],
                dst_ref=comm_ref.at[1],
                send_sem=send_sem,
                recv_sem=recv_sem,
                device_id=partner,
                device_id_type=pl.DeviceIdType.MESH,
            )
            rdma.start()
            rdma.wait()

            r = comm_ref[1]
            acc2 = r[:, :, :D]
            m2 = r[:, :, D]
            l2 = r[:, :, 2 * D]

            m12 = jnp.maximum(m, m2)
            a1 = jnp.exp(m - m12)
            a2 = jnp.exp(m2 - m12)
            lsum = a1 * l + a2 * l2
            o = (a1[:, :, None] * acc + a2[:, :, None] * acc2) \
                / lsum[:, :, None]
            out_ref[:, :, :, :] = o[:, None, :, :]

    return pl.pallas_call(
        body,
        grid=(nsteps,),
        out_shape=jax.ShapeDtypeStruct((B, 1, H, D), jnp.float32),
        in_specs=[
            pl.BlockSpec((B, 1, H, D), lambda c: (0, 0, 0, 0)),
            pl.BlockSpec((CP, BS, H, D), lambda c: (c, 0, 0, 0)),
            pl.BlockSpec((CP, BS, H, D), lambda c: (c, 0, 0, 0)),
            pl.BlockSpec((B, NB), lambda c: (0, 0)),
            pl.BlockSpec((B, 1), lambda c: (0, 0)),
        ],
        out_specs=pl.BlockSpec((B, 1, H, D), lambda c: (0, 0, 0, 0)),
        scratch_shapes=[
            pltpu.VMEM((2, B, H, 3 * D), jnp.float32),
            pltpu.VMEM((H, B, D), jnp.float32),
            pltpu.VMEM((H, B, 1), jnp.float32),
            pltpu.VMEM((H, B, 1), jnp.float32),
            pltpu.SemaphoreType.DMA,
            pltpu.SemaphoreType.DMA,
        ],
        compiler_params=pltpu.CompilerParams(collective_id=0),
    )(Q, K, V, bt, lens.reshape(B, 1))


# device time: 63381 ns/iter; 1.3937x vs baseline; 1.3937x over previous
import jax
import jax.numpy as jnp
from jax import lax
from jax.experimental import pallas as pl
from jax.experimental.pallas import tpu as pltpu

B, H, D, BS = 8, 8, 128, 16
NB = 512
CP = 64
CK = CP * BS

NEG_INF = -1e30


def kernel(Q, K, V, bt, lens):
    npages = K.shape[0]
    nchunks = npages // CP
    nt = nchunks * H

    def body(q_ref, k_hbm, v_hbm, bt_ref, lens_ref, out_ref,
             comm_ref, kbuf, vbuf, ksem, vsem, send_sem, recv_sem):
        my_x = lax.axis_index("x")
        my_y = lax.axis_index("y")
        my_z = lax.axis_index("z")
        partner = (my_x, 1 - my_y, my_z)

        def start_copy(t):
            c, h = divmod(t, H)
            slot = t % 2
            kc = pltpu.make_async_copy(
                k_hbm.at[c * CP:(c + 1) * CP, :, h, :],
                kbuf.at[slot], ksem.at[slot])
            vc = pltpu.make_async_copy(
                v_hbm.at[c * CP:(c + 1) * CP, :, h, :],
                vbuf.at[slot], vsem.at[slot])
            kc.start()
            vc.start()
            return kc, vc

        copies = [start_copy(0)]

        page_off = my_y * npages
        bt_v = bt_ref[:, :]
        lens_v = lens_ref[:, :]
        slot_i = lax.broadcasted_iota(jnp.int32, (B, NB, 1), 1)
        valid = slot_i < lens_v[:, :, None]
        pids = lax.broadcasted_iota(jnp.int32, (B, NB, npages), 2) + page_off
        match = jnp.logical_and(bt_v[:, :, None] == pids, valid)
        counts = jnp.sum(match.astype(jnp.float32), axis=1)
        w_all = jnp.repeat(counts, BS, axis=1)

        scale = D ** -0.5
        q_all = q_ref[:, 0, :, :]

        m = [None] * H
        l = [None] * H
        acc = [None] * H
        for t in range(nt):
            if t + 1 < nt:
                copies.append(start_copy(t + 1))
            c, h = divmod(t, H)
            slot = t % 2
            kc, vc = copies[t]
            kc.wait()
            vc.wait()
            kh = kbuf[slot]
            vh = vbuf[slot]
            qh = q_all[:, h, :]
            w = w_all[:, c * CK:(c + 1) * CK]
            s = lax.dot_general(
                qh, kh.reshape(CK, D), (((1,), (1,)), ((), ())),
                preferred_element_type=jnp.float32) * scale
            m_c = jnp.max(s, axis=1, keepdims=True)
            if m[h] is None:
                m_new = m_c
                alpha = None
            else:
                m_new = jnp.maximum(m[h], m_c)
                alpha = jnp.exp(m[h] - m_new)
            e = jnp.exp(s - m_new) * w
            pv = lax.dot_general(
                e, vh.reshape(CK, D), (((1,), (0,)), ((), ())),
                preferred_element_type=jnp.float32)
            l_c = jnp.sum(e, axis=1, keepdims=True)
            if alpha is None:
                acc[h] = pv
                l[h] = l_c
            else:
                acc[h] = alpha * acc[h] + pv
                l[h] = alpha * l[h] + l_c
            m[h] = m_new

        accs = jnp.stack(acc, axis=1)
        ms = jnp.concatenate(m, axis=1)
        ls = jnp.concatenate(l, axis=1)

        comm_ref[0] = jnp.concatenate(
            [accs,
             jnp.broadcast_to(ms[:, :, None], (B, H, D)),
             jnp.broadcast_to(ls[:, :, None], (B, H, D))], axis=2)

        barrier = pltpu.get_barrier_semaphore()
        pl.semaphore_signal(barrier, inc=1, device_id=partner,
                            device_id_type=pl.DeviceIdType.MESH)
        pl.semaphore_wait(barrier, 1)

        rdma = pltpu.make_async_remote_copy(
            src_ref=comm_ref.at[0],
            dst_ref=comm_ref.at[1],
            send_sem=send_sem,
            recv_sem=recv_sem,
            device_id=partner,
            device_id_type=pl.DeviceIdType.MESH,
        )
        rdma.start()
        rdma.wait()

        r = comm_ref[1]
        acc2 = r[:, :, :D]
        m2 = r[:, :, D]
        l2 = r[:, :, 2 * D]

        m12 = jnp.maximum(ms, m2)
        a1 = jnp.exp(ms - m12)
        a2 = jnp.exp(m2 - m12)
        lsum = a1 * ls + a2 * l2
        o = (a1[:, :, None] * accs + a2[:, :, None] * acc2) \
            / lsum[:, :, None]
        out_ref[:, :, :, :] = o[:, None, :, :]

    return pl.pallas_call(
        body,
        out_shape=jax.ShapeDtypeStruct((B, 1, H, D), jnp.float32),
        in_specs=[
            pl.BlockSpec(memory_space=pltpu.VMEM),
            pl.BlockSpec(memory_space=pltpu.MemorySpace.HBM),
            pl.BlockSpec(memory_space=pltpu.MemorySpace.HBM),
            pl.BlockSpec(memory_space=pltpu.VMEM),
            pl.BlockSpec(memory_space=pltpu.VMEM),
        ],
        out_specs=pl.BlockSpec(memory_space=pltpu.VMEM),
        scratch_shapes=[
            pltpu.VMEM((2, B, H, 3 * D), jnp.float32),
            pltpu.VMEM((2, CP, BS, D), jnp.float32),
            pltpu.VMEM((2, CP, BS, D), jnp.float32),
            pltpu.SemaphoreType.DMA((2,)),
            pltpu.SemaphoreType.DMA((2,)),
            pltpu.SemaphoreType.DMA,
            pltpu.SemaphoreType.DMA,
        ],
        compiler_params=pltpu.CompilerParams(collective_id=0),
    )(Q, K, V, bt, lens.reshape(B, 1))


# device time: 39488 ns/iter; 2.2370x vs baseline; 1.6051x over previous
import jax
import jax.numpy as jnp
from jax import lax
from jax.experimental import pallas as pl
from jax.experimental.pallas import tpu as pltpu

B, H, D, BS = 8, 8, 128, 16
NB = 512
CP = 64
CK = CP * BS
NSLOT = 4

NEG_INF = -1e30


def kernel(Q, K, V, bt, lens):
    npages = K.shape[0]
    nchunks = npages // CP
    nt = nchunks * H

    def body(q_ref, k_hbm, v_hbm, bt_ref, lens_ref, out_ref,
             comm_ref, kbuf, vbuf, ksem, vsem, send_sem, recv_sem):
        my_x = lax.axis_index("x")
        my_y = lax.axis_index("y")
        my_z = lax.axis_index("z")
        partner = (my_x, 1 - my_y, my_z)

        def start_copy(t):
            c, h = divmod(t, H)
            slot = t % NSLOT
            kc = pltpu.make_async_copy(
                k_hbm.at[c * CP:(c + 1) * CP, :, h, :],
                kbuf.at[slot], ksem.at[slot])
            vc = pltpu.make_async_copy(
                v_hbm.at[c * CP:(c + 1) * CP, :, h, :],
                vbuf.at[slot], vsem.at[slot])
            kc.start()
            vc.start()
            return kc, vc

        copies = [start_copy(t) for t in range(NSLOT - 1)]

        page_off = my_y * npages
        bt_v = bt_ref[:, :]
        lens_v = lens_ref[:, :]
        slot_i = lax.broadcasted_iota(jnp.int32, (B, NB, 1), 1)
        valid = slot_i < lens_v[:, :, None]
        pids = lax.broadcasted_iota(jnp.int32, (B, NB, npages), 2) + page_off
        match = jnp.logical_and(bt_v[:, :, None] == pids, valid)
        counts = jnp.sum(match.astype(jnp.float32), axis=1)
        w_all = jnp.repeat(counts, BS, axis=1)

        scale = D ** -0.5
        q_all = q_ref[:, 0, :, :]

        m = [None] * H
        l = [None] * H
        acc = [None] * H
        for t in range(nt):
            if t + NSLOT - 1 < nt:
                copies.append(start_copy(t + NSLOT - 1))
            c, h = divmod(t, H)
            slot = t % NSLOT
            kc, vc = copies[t]
            kc.wait()
            vc.wait()
            kh = kbuf[slot]
            vh = vbuf[slot]
            qh = q_all[:, h, :]
            w = w_all[:, c * CK:(c + 1) * CK]
            s = lax.dot_general(
                qh, kh.reshape(CK, D), (((1,), (1,)), ((), ())),
                preferred_element_type=jnp.float32) * scale
            m_c = jnp.max(s, axis=1, keepdims=True)
            if m[h] is None:
                m_new = m_c
                alpha = None
            else:
                m_new = jnp.maximum(m[h], m_c)
                alpha = jnp.exp(m[h] - m_new)
            e = jnp.exp(s - m_new) * w
            pv = lax.dot_general(
                e, vh.reshape(CK, D), (((1,), (0,)), ((), ())),
                preferred_element_type=jnp.float32)
            l_c = jnp.sum(e, axis=1, keepdims=True)
            if alpha is None:
                acc[h] = pv
                l[h] = l_c
            else:
                acc[h] = alpha * acc[h] + pv
                l[h] = alpha * l[h] + l_c
            m[h] = m_new

        accs = jnp.stack(acc, axis=1)
        ms = jnp.concatenate(m, axis=1)
        ls = jnp.concatenate(l, axis=1)

        comm_ref[0] = jnp.concatenate(
            [accs,
             jnp.broadcast_to(ms[:, :, None], (B, H, D)),
             jnp.broadcast_to(ls[:, :, None], (B, H, D))], axis=2)

        barrier = pltpu.get_barrier_semaphore()
        pl.semaphore_signal(barrier, inc=1, device_id=partner,
                            device_id_type=pl.DeviceIdType.MESH)
        pl.semaphore_wait(barrier, 1)

        rdma = pltpu.make_async_remote_copy(
            src_ref=comm_ref.at[0],
            dst_ref=comm_ref.at[1],
            send_sem=send_sem,
            recv_sem=recv_sem,
            device_id=partner,
            device_id_type=pl.DeviceIdType.MESH,
        )
        rdma.start()
        rdma.wait()

        r = comm_ref[1]
        acc2 = r[:, :, :D]
        m2 = r[:, :, D]
        l2 = r[:, :, 2 * D]

        m12 = jnp.maximum(ms, m2)
        a1 = jnp.exp(ms - m12)
        a2 = jnp.exp(m2 - m12)
        lsum = a1 * ls + a2 * l2
        o = (a1[:, :, None] * accs + a2[:, :, None] * acc2) \
            / lsum[:, :, None]
        out_ref[:, :, :, :] = o[:, None, :, :]

    return pl.pallas_call(
        body,
        out_shape=jax.ShapeDtypeStruct((B, 1, H, D), jnp.float32),
        in_specs=[
            pl.BlockSpec(memory_space=pltpu.VMEM),
            pl.BlockSpec(memory_space=pltpu.MemorySpace.HBM),
            pl.BlockSpec(memory_space=pltpu.MemorySpace.HBM),
            pl.BlockSpec(memory_space=pltpu.VMEM),
            pl.BlockSpec(memory_space=pltpu.VMEM),
        ],
        out_specs=pl.BlockSpec(memory_space=pltpu.VMEM),
        scratch_shapes=[
            pltpu.VMEM((2, B, H, 3 * D), jnp.float32),
            pltpu.VMEM((NSLOT, CP, BS, D), jnp.float32),
            pltpu.VMEM((NSLOT, CP, BS, D), jnp.float32),
            pltpu.SemaphoreType.DMA((NSLOT,)),
            pltpu.SemaphoreType.DMA((NSLOT,)),
            pltpu.SemaphoreType.DMA,
            pltpu.SemaphoreType.DMA,
        ],
        compiler_params=pltpu.CompilerParams(collective_id=0),
    )(Q, K, V, bt, lens.reshape(B, 1))
